# baseline (speedup 1.0000x reference)
import functools

import jax
import jax.numpy as jnp
from jax import lax
from jax.experimental import pallas as pl
from jax.experimental.pallas import tpu as pltpu
from jax.experimental.pallas import tpu_sc as plsc

N = 10000
D = 128
NC = 2
NS = 16
NW = NC * NS
NPAD = 10240
BPW = NPAD // NW
CH = 4
CB = BPW // CH

BR = 200
GI = N // BR

_sc_mesh = plsc.VectorSubcoreMesh(
    core_axis_name="c", subcore_axis_name="s", num_cores=NC, num_subcores=NS)


@functools.partial(
    pl.kernel,
    out_type=jax.ShapeDtypeStruct((N, D), jnp.float32),
    mesh=_sc_mesh,
    scratch_types=[
        pltpu.VMEM((CH, CB), jnp.int32),
        pltpu.VMEM((BPW, D), jnp.float32),
        pltpu.SemaphoreType.DMA,
        pltpu.SemaphoreType.DMA,
    ],
)
def _sc_gather(idx_hbm, table_hbm, out_hbm, idx_v, rows_v, sem, osem):
  w = lax.axis_index("s") * NC + lax.axis_index("c")
  base = jnp.minimum(w * BPW, N - BPW)
  pltpu.sync_copy(idx_hbm.at[w], idx_v)
  copies = [
      pltpu.async_copy(
          table_hbm.at[idx_v.at[j]], rows_v.at[pl.ds(j * CB, CB)], sem)
      for j in range(CH)
  ]
  outs = []
  for j, c in enumerate(copies):
    c.wait()
    outs.append(pltpu.async_copy(
        rows_v.at[pl.ds(j * CB, CB)],
        out_hbm.at[pl.ds(base + j * CB, CB)], osem))
  for o in outs:
    o.wait()


def _tc_body(adj_ref, ue_ref, w1_ref, b1_ref, w2_ref, b2_ref, lw1_ref,
             lb1_ref, lw2_ref, lb2_ref, x_ref, s1_ref, s2_ref):
  i = pl.program_id(0)

  @pl.when(i == 0)
  def _():
    s1_ref[...] = jnp.dot(ue_ref[...], w1_ref[...],
                          preferred_element_type=jnp.float32)

  @pl.when(i < GI)
  def _():
    h = jnp.dot(adj_ref[...], s1_ref[...], preferred_element_type=jnp.float32)
    r = jnp.maximum(h + b1_ref[...], 0.0)
    s2_ref[pl.ds(i * BR, BR), :] = jnp.dot(
        r, w2_ref[...], preferred_element_type=jnp.float32)

  @pl.when(i >= GI)
  def _():
    h2 = jnp.dot(adj_ref[...], s2_ref[...],
                 preferred_element_type=jnp.float32) + b2_ref[...]
    y = jnp.dot(h2, lw1_ref[...],
                preferred_element_type=jnp.float32) + lb1_ref[...]
    x_ref[...] = jnp.dot(y, lw2_ref[...],
                         preferred_element_type=jnp.float32) + lb2_ref[...]


_tc_call = pl.pallas_call(
    _tc_body,
    grid=(2 * GI,),
    in_specs=[
        pl.BlockSpec((BR, N), lambda i: (lax.rem(i, GI), 0)),
        pl.BlockSpec((N, D), lambda i: (0, 0)),
        pl.BlockSpec((D, D), lambda i: (0, 0)),
        pl.BlockSpec((1, D), lambda i: (0, 0)),
        pl.BlockSpec((D, D), lambda i: (0, 0)),
        pl.BlockSpec((1, D), lambda i: (0, 0)),
        pl.BlockSpec((D, 16), lambda i: (0, 0)),
        pl.BlockSpec((1, 16), lambda i: (0, 0)),
        pl.BlockSpec((16, 1), lambda i: (0, 0)),
        pl.BlockSpec((1, 1), lambda i: (0, 0)),
    ],
    out_specs=pl.BlockSpec(
        (BR, 1), lambda i: (jnp.where(i >= GI, i - GI, 0), 0)),
    out_shape=jax.ShapeDtypeStruct((N, 1), jnp.float32),
    scratch_shapes=[
        pltpu.VMEM((N, D), jnp.float32),
        pltpu.VMEM((N, D), jnp.float32),
    ],
    compiler_params=pltpu.CompilerParams(
        dimension_semantics=("arbitrary",)),
)


@jax.jit
def kernel(features, adj, emb_table, W1, b1, W2, b2, lw1, lb1, lw2, lb2):
  feat = features.astype(jnp.int32)
  starts = jnp.minimum(jnp.arange(NW, dtype=jnp.int32) * BPW, N - BPW)
  idx3 = feat[starts[:, None] + jnp.arange(BPW, dtype=jnp.int32)[None, :]]
  idx3 = idx3.reshape(NW, CH, CB)
  user_emb = _sc_gather(idx3, emb_table)

  x = _tc_call(adj, user_emb, W1, b1.reshape(1, D), W2, b2.reshape(1, D),
               lw1, lb1.reshape(1, 16), lw2, lb2.reshape(1, 1))
  return (x, user_emb)


# speedup vs baseline: 1.0261x; 1.0261x over previous
import functools

import jax
import jax.numpy as jnp
from jax import lax
from jax.experimental import pallas as pl
from jax.experimental.pallas import tpu as pltpu
from jax.experimental.pallas import tpu_sc as plsc

N = 10000
D = 128
NC = 2
NS = 16
NW = NC * NS
NPAD = 10240
BPW = NPAD // NW
CH = 4
CB = BPW // CH

BR = 400
GI = N // BR

_sc_mesh = plsc.VectorSubcoreMesh(
    core_axis_name="c", subcore_axis_name="s", num_cores=NC, num_subcores=NS)


@functools.partial(
    pl.kernel,
    out_type=jax.ShapeDtypeStruct((N, D), jnp.float32),
    mesh=_sc_mesh,
    scratch_types=[
        pltpu.VMEM((CH, CB), jnp.int32),
        pltpu.VMEM((BPW, D), jnp.float32),
        pltpu.SemaphoreType.DMA,
        pltpu.SemaphoreType.DMA,
    ],
)
def _sc_gather(idx_hbm, table_hbm, out_hbm, idx_v, rows_v, sem, osem):
  w = lax.axis_index("s") * NC + lax.axis_index("c")
  base = jnp.minimum(w * BPW, N - BPW)
  pltpu.sync_copy(idx_hbm.at[w], idx_v)
  copies = [
      pltpu.async_copy(
          table_hbm.at[idx_v.at[j]], rows_v.at[pl.ds(j * CB, CB)], sem)
      for j in range(CH)
  ]
  outs = []
  for j, c in enumerate(copies):
    c.wait()
    outs.append(pltpu.async_copy(
        rows_v.at[pl.ds(j * CB, CB)],
        out_hbm.at[pl.ds(base + j * CB, CB)], osem))
  for o in outs:
    o.wait()


def _tc_body(adj_ref, ue_ref, w1_ref, b1_ref, w2_ref, b2_ref, lw1_ref,
             lb1_ref, lw2_ref, lb2_ref, x_ref, s1_ref, s2_ref):
  i = pl.program_id(0)

  @pl.when(i == 0)
  def _():
    s1_ref[...] = jnp.dot(ue_ref[...], w1_ref[...],
                          preferred_element_type=jnp.float32)

  @pl.when(i < GI)
  def _():
    h = jnp.dot(adj_ref[...], s1_ref[...], preferred_element_type=jnp.float32)
    r = jnp.maximum(h + b1_ref[...], 0.0)
    s2_ref[pl.ds(i * BR, BR), :] = jnp.dot(
        r, w2_ref[...], preferred_element_type=jnp.float32)

  @pl.when(i >= GI)
  def _():
    h2 = jnp.dot(adj_ref[...], s2_ref[...],
                 preferred_element_type=jnp.float32) + b2_ref[...]
    y = jnp.dot(h2, lw1_ref[...],
                preferred_element_type=jnp.float32) + lb1_ref[...]
    x_ref[...] = jnp.dot(y, lw2_ref[...],
                         preferred_element_type=jnp.float32) + lb2_ref[...]


_tc_call = pl.pallas_call(
    _tc_body,
    grid=(2 * GI,),
    in_specs=[
        pl.BlockSpec((BR, N), lambda i: (lax.rem(i, GI), 0)),
        pl.BlockSpec((N, D), lambda i: (0, 0)),
        pl.BlockSpec((D, D), lambda i: (0, 0)),
        pl.BlockSpec((1, D), lambda i: (0, 0)),
        pl.BlockSpec((D, D), lambda i: (0, 0)),
        pl.BlockSpec((1, D), lambda i: (0, 0)),
        pl.BlockSpec((D, 16), lambda i: (0, 0)),
        pl.BlockSpec((1, 16), lambda i: (0, 0)),
        pl.BlockSpec((16, 1), lambda i: (0, 0)),
        pl.BlockSpec((1, 1), lambda i: (0, 0)),
    ],
    out_specs=pl.BlockSpec(
        (BR, 1), lambda i: (jnp.where(i >= GI, i - GI, 0), 0)),
    out_shape=jax.ShapeDtypeStruct((N, 1), jnp.float32),
    scratch_shapes=[
        pltpu.VMEM((N, D), jnp.float32),
        pltpu.VMEM((N, D), jnp.float32),
    ],
    compiler_params=pltpu.CompilerParams(
        dimension_semantics=("arbitrary",)),
)


@jax.jit
def kernel(features, adj, emb_table, W1, b1, W2, b2, lw1, lb1, lw2, lb2):
  feat = features.astype(jnp.int32)
  starts = jnp.minimum(jnp.arange(NW, dtype=jnp.int32) * BPW, N - BPW)
  idx3 = feat[starts[:, None] + jnp.arange(BPW, dtype=jnp.int32)[None, :]]
  idx3 = idx3.reshape(NW, CH, CB)
  user_emb = _sc_gather(idx3, emb_table)

  x = _tc_call(adj, user_emb, W1, b1.reshape(1, D), W2, b2.reshape(1, D),
               lw1, lb1.reshape(1, 16), lw2, lb2.reshape(1, 1))
  return (x, user_emb)


# speedup vs baseline: 1.0535x; 1.0267x over previous
import functools

import jax
import jax.numpy as jnp
from jax import lax
from jax.experimental import pallas as pl
from jax.experimental.pallas import tpu as pltpu
from jax.experimental.pallas import tpu_sc as plsc

N = 10000
D = 128
NC = 2
NS = 16
NW = NC * NS
NPAD = 10240
BPW = NPAD // NW
CH = 4
CB = BPW // CH

BR = 400
GI = N // BR

_sc_mesh = plsc.VectorSubcoreMesh(
    core_axis_name="c", subcore_axis_name="s", num_cores=NC, num_subcores=NS)


@functools.partial(
    pl.kernel,
    out_type=jax.ShapeDtypeStruct((N, D), jnp.float32),
    mesh=_sc_mesh,
    scratch_types=[
        pltpu.VMEM((CH, CB), jnp.int32),
        pltpu.VMEM((BPW, D), jnp.float32),
        pltpu.SemaphoreType.DMA,
        pltpu.SemaphoreType.DMA,
    ],
)
def _sc_gather(idx_hbm, table_hbm, out_hbm, idx_v, rows_v, sem, osem):
  w = lax.axis_index("s") * NC + lax.axis_index("c")
  base = jnp.minimum(w * BPW, N - BPW)
  pltpu.sync_copy(idx_hbm.at[w], idx_v)
  copies = [
      pltpu.async_copy(
          table_hbm.at[idx_v.at[j]], rows_v.at[pl.ds(j * CB, CB)], sem)
      for j in range(CH)
  ]
  outs = []
  for j, c in enumerate(copies):
    c.wait()
    outs.append(pltpu.async_copy(
        rows_v.at[pl.ds(j * CB, CB)],
        out_hbm.at[pl.ds(base + j * CB, CB)], osem))
  for o in outs:
    o.wait()


def _tc_body(adj_ref, ue_ref, w1_ref, b1_ref, w2_ref, b2_ref, lw1_ref,
             lb1_ref, lw2_ref, lb2_ref, x_ref, s1_ref, s2_ref):
  i = pl.program_id(0)

  @pl.when(i == 0)
  def _():
    s1_ref[...] = jnp.dot(ue_ref[...], w1_ref[...],
                          preferred_element_type=jnp.float32)

  @pl.when(i < GI)
  def _():
    h = jnp.dot(adj_ref[...], s1_ref[...], preferred_element_type=jnp.float32)
    r = jnp.maximum(h + b1_ref[...], 0.0)
    s2_ref[pl.ds(i * BR, BR), :] = jnp.dot(
        r, w2_ref[...], preferred_element_type=jnp.float32)

  @pl.when(i >= GI)
  def _():
    h2 = jnp.dot(adj_ref[...], s2_ref[...],
                 preferred_element_type=jnp.float32) + b2_ref[...]
    y = jnp.dot(h2, lw1_ref[...],
                preferred_element_type=jnp.float32) + lb1_ref[...]
    x_ref[...] = jnp.dot(y, lw2_ref[...],
                         preferred_element_type=jnp.float32) + lb2_ref[...]


_tc_call = pl.pallas_call(
    _tc_body,
    grid=(2 * GI,),
    in_specs=[
        pl.BlockSpec((BR, N), lambda i: (lax.rem(i, GI), 0)),
        pl.BlockSpec((N, D), lambda i: (0, 0)),
        pl.BlockSpec((D, D), lambda i: (0, 0)),
        pl.BlockSpec((1, D), lambda i: (0, 0)),
        pl.BlockSpec((D, D), lambda i: (0, 0)),
        pl.BlockSpec((1, D), lambda i: (0, 0)),
        pl.BlockSpec((D, 16), lambda i: (0, 0)),
        pl.BlockSpec((1, 16), lambda i: (0, 0)),
        pl.BlockSpec((16, 1), lambda i: (0, 0)),
        pl.BlockSpec((1, 1), lambda i: (0, 0)),
    ],
    out_specs=pl.BlockSpec(
        (BR, 1), lambda i: (jnp.where(i >= GI, i - GI, 0), 0)),
    out_shape=jax.ShapeDtypeStruct((N, 1), jnp.float32),
    scratch_shapes=[
        pltpu.VMEM((N, D), jnp.float32),
        pltpu.VMEM((N, D), jnp.float32),
    ],
    compiler_params=pltpu.CompilerParams(
        dimension_semantics=("arbitrary",)),
)


@jax.jit
def kernel(features, adj, emb_table, W1, b1, W2, b2, lw1, lb1, lw2, lb2):
  feat = features.astype(jnp.int32)
  idx3 = jnp.concatenate([feat[:(NW - 1) * BPW], feat[N - BPW:]])
  idx3 = idx3.reshape(NW, CH, CB)
  user_emb = _sc_gather(idx3, emb_table)

  x = _tc_call(adj, user_emb, W1, b1.reshape(1, D), W2, b2.reshape(1, D),
               lw1, lb1.reshape(1, 16), lw2, lb2.reshape(1, 1))
  return (x, user_emb)


# speedup vs baseline: 1.0560x; 1.0023x over previous
import functools

import jax
import jax.numpy as jnp
from jax import lax
from jax.experimental import pallas as pl
from jax.experimental.pallas import tpu as pltpu
from jax.experimental.pallas import tpu_sc as plsc

N = 10000
D = 128
NC = 2
NS = 16
NW = NC * NS
NPAD = 10240
BPW = NPAD // NW
CH = 4
CB = BPW // CH

BR = 400
GI = N // BR

_sc_mesh = plsc.VectorSubcoreMesh(
    core_axis_name="c", subcore_axis_name="s", num_cores=NC, num_subcores=NS)


@functools.partial(
    pl.kernel,
    out_type=jax.ShapeDtypeStruct((N, D), jnp.float32),
    mesh=_sc_mesh,
    scratch_types=[
        pltpu.VMEM((BPW,), jnp.int32),
        pltpu.VMEM((BPW, D), jnp.float32),
        pltpu.SemaphoreType.DMA,
        pltpu.SemaphoreType.DMA,
    ],
)
def _sc_gather(idx_hbm, table_hbm, out_hbm, idx_v, rows_v, sem, osem):
  w = lax.axis_index("s") * NC + lax.axis_index("c")
  base = jnp.minimum(w * BPW, N - BPW)
  pltpu.sync_copy(idx_hbm.at[pl.ds(base, BPW)], idx_v)
  copies = [
      pltpu.async_copy(
          table_hbm.at[idx_v.at[pl.ds(j * CB, CB)]],
          rows_v.at[pl.ds(j * CB, CB)], sem)
      for j in range(CH)
  ]
  outs = []
  for j, c in enumerate(copies):
    c.wait()
    outs.append(pltpu.async_copy(
        rows_v.at[pl.ds(j * CB, CB)],
        out_hbm.at[pl.ds(base + j * CB, CB)], osem))
  for o in outs:
    o.wait()


def _tc_body(adj_ref, ue_ref, w1_ref, b1_ref, w2_ref, b2_ref, lw1_ref,
             lb1_ref, lw2_ref, lb2_ref, x_ref, s1_ref, s2_ref):
  i = pl.program_id(0)

  @pl.when(i == 0)
  def _():
    s1_ref[...] = jnp.dot(ue_ref[...], w1_ref[...],
                          preferred_element_type=jnp.float32)

  @pl.when(i < GI)
  def _():
    h = jnp.dot(adj_ref[...], s1_ref[...], preferred_element_type=jnp.float32)
    r = jnp.maximum(h + b1_ref[...], 0.0)
    s2_ref[pl.ds(i * BR, BR), :] = jnp.dot(
        r, w2_ref[...], preferred_element_type=jnp.float32)

  @pl.when(i >= GI)
  def _():
    h2 = jnp.dot(adj_ref[...], s2_ref[...],
                 preferred_element_type=jnp.float32) + b2_ref[...]
    y = jnp.dot(h2, lw1_ref[...],
                preferred_element_type=jnp.float32) + lb1_ref[...]
    x_ref[...] = jnp.dot(y, lw2_ref[...],
                         preferred_element_type=jnp.float32) + lb2_ref[...]


_tc_call = pl.pallas_call(
    _tc_body,
    grid=(2 * GI,),
    in_specs=[
        pl.BlockSpec((BR, N), lambda i: (lax.rem(i, GI), 0)),
        pl.BlockSpec((N, D), lambda i: (0, 0)),
        pl.BlockSpec((D, D), lambda i: (0, 0)),
        pl.BlockSpec((1, D), lambda i: (0, 0)),
        pl.BlockSpec((D, D), lambda i: (0, 0)),
        pl.BlockSpec((1, D), lambda i: (0, 0)),
        pl.BlockSpec((D, 16), lambda i: (0, 0)),
        pl.BlockSpec((1, 16), lambda i: (0, 0)),
        pl.BlockSpec((16, 1), lambda i: (0, 0)),
        pl.BlockSpec((1, 1), lambda i: (0, 0)),
    ],
    out_specs=pl.BlockSpec(
        (BR, 1), lambda i: (jnp.where(i >= GI, i - GI, 0), 0)),
    out_shape=jax.ShapeDtypeStruct((N, 1), jnp.float32),
    scratch_shapes=[
        pltpu.VMEM((N, D), jnp.float32),
        pltpu.VMEM((N, D), jnp.float32),
    ],
    compiler_params=pltpu.CompilerParams(
        dimension_semantics=("arbitrary",)),
)


@jax.jit
def kernel(features, adj, emb_table, W1, b1, W2, b2, lw1, lb1, lw2, lb2):
  user_emb = _sc_gather(features.astype(jnp.int32), emb_table)

  x = _tc_call(adj, user_emb, W1, b1.reshape(1, D), W2, b2.reshape(1, D),
               lw1, lb1.reshape(1, 16), lw2, lb2.reshape(1, 1))
  return (x, user_emb)
